# baseline (speedup 1.0000x reference)
import jax
import jax.numpy as jnp
from jax import lax
from jax.experimental import pallas as pl
from jax.experimental.pallas import tpu as pltpu
from jax.experimental.pallas import tpu_sc as plsc

_NC, _NS, _L = 2, 16, 16
_NW = _NC * _NS

_B, _C, _H, _W = 8, 256, 128, 128
_K = 128
_HW = _H * _W
_NP = _B * _K
_PPW = _NP // _NW
_NCHUNK = _PPW // _L
_CB = _C // _L

_P = (-1.72080611e-02, 8.17268084e-02, -1.88782674e-01, 3.14590535e-01,
      -4.96977911e-01, 9.99792436e-01, 3.50755205e-06)


def _softplus_neg(z):
    e = jnp.exp(z)
    acc = jnp.full_like(e, _P[0])
    for c in _P[1:]:
        acc = acc * e + c
    return acc


def _lane(vec, i):
    return jnp.squeeze(lax.slice(vec, (i,), (i + 1,)))


def _sc_body(table_hbm, ind_hbm, mask_hbm, tgt_hbm, out_hbm,
             ind_v, mask_v, tgt_v, idx_v, rows_v, out_v,
             sem0, sem1, sem_in):
    wid = lax.axis_index("s") * _NC + lax.axis_index("c")
    base_pair = wid * _PPW
    pltpu.sync_copy(ind_hbm.at[pl.ds(base_pair, _PPW)], ind_v)
    pltpu.sync_copy(mask_hbm.at[pl.ds(base_pair, _PPW)], mask_v)
    tgt_cp = pltpu.async_copy(
        tgt_hbm.at[pl.ds(base_pair * _C, _PPW * _C)], tgt_v, sem_in)

    iot = lax.iota(jnp.int32, _L)
    iothw = iot * _HW
    sems = (sem0, sem1)

    live, bases = [], []
    for chunk in range(_NCHUNK):
        iv = ind_v[pl.ds(chunk * _L, _L)]
        mv = mask_v[pl.ds(chunk * _L, _L)]
        q = base_pair + chunk * _L + iot
        b = lax.shift_right_logical(q, 7)
        base = b * (_C * _HW) + iv
        for i in range(_L):
            live.append(_lane(mv, i) != 0)
            bases.append(_lane(base, i))

    def _fire(p):
        @pl.when(live[p])
        def _(p=p):
            bp = bases[p]
            for cb in range(_CB):
                idx_v[pl.ds(p * _C + cb * _L, _L)] = (
                    bp + cb * _L * _HW) + iothw
            for h in range(2):
                pltpu.async_copy(
                    table_hbm.at[idx_v.at[pl.ds(p * _C + h * 128, 128)]],
                    rows_v.at[pl.ds(p * _C + h * 128, 128)],
                    sems[p // _L])

    def _drain(p):
        @pl.when(live[p])
        def _(p=p):
            for h in range(2):
                pltpu.make_async_copy(
                    table_hbm.at[idx_v.at[pl.ds(p * _C + h * 128, 128)]],
                    rows_v.at[pl.ds(p * _C + h * 128, 128)],
                    sems[p // _L]).wait()

    def _compute(p):
        @pl.when(live[p])
        def _(p=p):
            def _accum(cb, carry, p=p):
                pred = rows_v[pl.ds(p * _C + cb * _L, _L)]
                tgt = tgt_v[pl.ds(p * _C + cb * _L, _L)]
                bce = (jnp.maximum(pred, 0.0) - pred * tgt
                       + _softplus_neg(-jnp.abs(pred)))
                return carry + bce
            a = lax.fori_loop(0, _CB, _accum, jnp.zeros((_L,), jnp.float32))
            out_v[pl.ds(0, _L)] = out_v[pl.ds(0, _L)] + a

    for p in range(_PPW):
        _fire(p)
    out_v[pl.ds(0, _L)] = jnp.zeros((_L,), jnp.float32)
    tgt_cp.wait()
    for p in range(_L):
        _drain(p)
    for p in range(_L):
        _compute(p)
    for p in range(_L, _PPW):
        _drain(p)
    for p in range(_L, _PPW):
        _compute(p)

    n = jnp.int32(0)
    for p in range(_PPW):
        n = n + jnp.where(live[p], 1, 0)
    out_v[pl.ds(_L, _L)] = jnp.where(iot == 0, iot * 0.0 + n, 0.0)
    pltpu.sync_copy(out_v, out_hbm.at[wid])


@jax.jit
def kernel(output, mask, ind, target):
    table = output.reshape(_B * _C * _HW)
    ind_flat = ind.reshape(_NP).astype(jnp.int32)
    mask_flat = mask.reshape(_NP).astype(jnp.int32)
    tgt = target.reshape(_NP * _C)

    call = pl.kernel(
        _sc_body,
        out_type=jax.ShapeDtypeStruct((_NW, 2 * _L), jnp.float32),
        mesh=plsc.VectorSubcoreMesh(core_axis_name="c", subcore_axis_name="s",
                                    num_cores=_NC, num_subcores=_NS),
        scratch_types=[
            pltpu.VMEM((_PPW,), jnp.int32),
            pltpu.VMEM((_PPW,), jnp.int32),
            pltpu.VMEM((_PPW * _C,), jnp.float32),
            pltpu.VMEM((_PPW * _C,), jnp.int32),
            pltpu.VMEM((_PPW * _C,), jnp.float32),
            pltpu.VMEM((2 * _L,), jnp.float32),
            pltpu.SemaphoreType.DMA,
            pltpu.SemaphoreType.DMA,
            pltpu.SemaphoreType.DMA,
        ],
    )
    partials = call(table, ind_flat, mask_flat, tgt)
    loss_sum = jnp.sum(partials[:, :_L])
    denom = jnp.sum(partials[:, _L:]) * _C + 0.0001
    return loss_sum / denom
